# baseline (speedup 1.0000x reference)
import functools

import jax
import jax.numpy as jnp
from jax import lax
from jax.experimental import pallas as pl
from jax.experimental.pallas import tpu as pltpu
from jax.experimental.pallas import tpu_sc as plsc

_NC = 2
_NS = 16
_NW = _NC * _NS
_L = 16
_B = 128


def _dense_pre(ent, attf, valf, a_w, a_b2, W):
    n, kd = ent.shape
    vd = valf.shape[1]
    kh = kd // 2
    blk = 1000
    grid = n // blk

    def body(ent_ref, attf_ref, valf_ref, aw_ref, ab_ref, w_ref,
             u_ref, v_ref, a_out_ref, v_out_ref):
        aw = aw_ref[...]
        w = w_ref[...]
        zpad = jnp.zeros((kd, kd - 1), jnp.float32)
        rhs_u = jnp.concatenate([aw[:, :kd].T, zpad], axis=1)
        rhs_v = jnp.concatenate([aw[:, kd:].T, zpad], axis=1)
        resu = jnp.dot(ent_ref[...], rhs_u, preferred_element_type=jnp.float32)
        resv = jnp.dot(attf_ref[...], rhs_v, preferred_element_type=jnp.float32)
        u_ref[...] = resu[:, :1] + ab_ref[0, 0]
        v_ref[...] = resv[:, :1]
        a_out_ref[...] = jnp.dot(attf_ref[...], w[:kd, :],
                                 preferred_element_type=jnp.float32)
        v_out_ref[...] = jnp.dot(valf_ref[...], w[kd:, :],
                                 preferred_element_type=jnp.float32)

    out = pl.pallas_call(
        body,
        grid=(grid,),
        in_specs=[
            pl.BlockSpec((blk, kd), lambda i: (i, 0)),
            pl.BlockSpec((blk, kd), lambda i: (i, 0)),
            pl.BlockSpec((blk, vd), lambda i: (i, 0)),
            pl.BlockSpec((1, 2 * kd), lambda i: (0, 0)),
            pl.BlockSpec((1, 1), lambda i: (0, 0)),
            pl.BlockSpec((kd + vd, kd), lambda i: (0, 0)),
        ],
        out_specs=[
            pl.BlockSpec((blk, 1), lambda i: (i, 0)),
            pl.BlockSpec((blk, 1), lambda i: (i, 0)),
            pl.BlockSpec((blk, kd), lambda i: (i, 0)),
            pl.BlockSpec((blk, kd), lambda i: (i, 0)),
        ],
        out_shape=[
            jax.ShapeDtypeStruct((n, 1), jnp.float32),
            jax.ShapeDtypeStruct((n, 1), jnp.float32),
            jax.ShapeDtypeStruct((n, kd), jnp.float32),
            jax.ShapeDtypeStruct((n, kd), jnp.float32),
        ],
    )(ent, attf, valf, a_w, a_b2, W)
    u, v, a_tab, v_tab = out
    return u.reshape(n), v.reshape(n), a_tab, v_tab


def _sc_scores(h2, a2, u, v, n, real_rows):
    rows = h2.shape[0]
    rpt = rows // _NW
    mesh = plsc.VectorSubcoreMesh(core_axis_name="c", subcore_axis_name="s")

    @functools.partial(
        pl.kernel,
        out_type=(
            jax.ShapeDtypeStruct((rows, _B), jnp.float32),
            jax.ShapeDtypeStruct((_NC * n,), jnp.float32),
        ),
        mesh=mesh,
        scratch_types=[
            pltpu.VMEM((n,), jnp.float32),
            pltpu.VMEM((n,), jnp.float32),
            pltpu.VMEM((rpt, _B), jnp.int32),
            pltpu.VMEM((rpt, _B), jnp.int32),
            pltpu.VMEM((rpt, _B), jnp.float32),
            pltpu.VMEM_SHARED((n,), jnp.float32),
        ],
        compiler_params=pltpu.CompilerParams(needs_layout_passes=False),
    )
    def run(h2_hbm, a2_hbm, u_hbm, v_hbm, s2_hbm, rs_hbm,
            u_v, v_v, h_v, a_v, s_v, rs_sh):
        c = lax.axis_index("c")
        sid = lax.axis_index("s")
        wid = sid * _NC + c
        base = wid * rpt

        @pl.when(sid == 0)
        def _zero():
            def zloop(i, carry):
                u_v[pl.ds(i * _L, _L)] = jnp.zeros((_L,), jnp.float32)
                return carry
            lax.fori_loop(0, n // _L, zloop, 0)
            pltpu.sync_copy(u_v, rs_sh)

        pltpu.sync_copy(u_hbm, u_v)
        pltpu.sync_copy(v_hbm, v_v)
        pltpu.sync_copy(h2_hbm.at[pl.ds(base, rpt)], h_v)
        pltpu.sync_copy(a2_hbm.at[pl.ds(base, rpt)], a_v)
        plsc.subcore_barrier()

        def chunk(j, carry):
            @pl.when(base + j < real_rows)
            def _work():
                for k in range(_B // _L):
                    sl = pl.ds(k * _L, _L)
                    h16 = h_v[j, sl]
                    a16 = a_v[j, sl]
                    t = plsc.load_gather(u_v, [h16]) + plsc.load_gather(v_v, [a16])
                    t = jnp.where(t > 0, t, 0.2 * t)
                    s_v[j, sl] = jnp.exp(t)
                pltpu.sync_copy(s_v.at[j], rs_sh.at[h_v.at[j]], add=True)

            @pl.when(base + j >= real_rows)
            def _pad():
                for k in range(_B // _L):
                    s_v[j, pl.ds(k * _L, _L)] = jnp.zeros((_L,), jnp.float32)
            return carry

        lax.fori_loop(0, rpt, chunk, 0)
        pltpu.sync_copy(s_v, s2_hbm.at[pl.ds(base, rpt)])
        plsc.subcore_barrier()

        @pl.when(sid == 0)
        def _flush():
            pltpu.sync_copy(rs_sh, u_v)
            pltpu.sync_copy(u_v, rs_hbm.at[pl.ds(c * n, n)])

    return run(h2, a2, u, v)


_BA = 64
_GR = 8
_WIN = 16
_D = 2
_C1_SHARE = 0.45


def _sc_aggregate(h3, av3, sts, tvt, n, kd):
    ngrp = h3.shape[0]
    gpt2 = ngrp // _NS
    gpt1 = 2 * (int(gpt2 * _C1_SHARE) // 2)
    gpt0 = gpt2 - gpt1
    nwin0 = gpt0 // 2
    nwin1 = gpt1 // 2
    n_acc = _NS * 640 * ((n + _NS * 640 - 1) // (_NS * 640))
    zrows = n_acc // _NS
    mesh = plsc.VectorSubcoreMesh(core_axis_name="c", subcore_axis_name="s")

    @functools.partial(
        pl.kernel,
        out_type=jax.ShapeDtypeStruct((_NC, n_acc, kd), jnp.float32),
        mesh=mesh,
        scratch_types=[
            pltpu.VMEM((_GR, _BA), jnp.int32),
            pltpu.VMEM((_GR, _BA), jnp.int32),
            pltpu.VMEM((_GR, 2 * _BA), jnp.int32),
            pltpu.VMEM((_GR, 2 * _BA), jnp.int32),
            pltpu.VMEM((_GR, _BA), jnp.float32),
            pltpu.VMEM((_GR, _BA), jnp.float32),
            pltpu.VMEM((2 * _BA, kd), jnp.float32),
            pltpu.VMEM((2 * _BA, kd), jnp.float32),
            pltpu.VMEM_SHARED((n_acc, kd), jnp.float32),
            pltpu.SemaphoreType.DMA,
            pltpu.SemaphoreType.DMA,
            pltpu.SemaphoreType.DMA,
        ],
        compiler_params=pltpu.CompilerParams(needs_layout_passes=False),
    )
    def run(h3_hbm, av3_hbm, sts_hbm, tvt_hbm, out_hbm,
            hA, hB, avA, avB, stsA, stsB, rb0, rb1,
            acc_sh, g0, g1, stg):
        c = lax.axis_index("c")
        sid = lax.axis_index("s")
        gbase = jnp.where(c == 0, sid * gpt0, _NS * gpt0 + sid * gpt1)
        nwin = jnp.where(c == 0, nwin0, nwin1)

        H = (hA, hB)
        AV = (avA, avB)
        STS = (stsA, stsB)
        RB = (rb0, rb1)
        G = (g0, g1)

        def zfill(i, carry):
            for k in range(kd // _L):
                rb0[i, pl.ds(k * _L, _L)] = jnp.zeros((_L,), jnp.float32)
            return carry
        lax.fori_loop(0, 2 * _BA, zfill, 0)
        for q in range(zrows // (2 * _BA)):
            pltpu.sync_copy(
                rb0, acc_sh.at[pl.ds(sid * zrows + q * 2 * _BA, 2 * _BA)])

        pltpu.sync_copy(h3_hbm.at[gbase], hA)
        pltpu.sync_copy(av3_hbm.at[gbase], avA)
        pltpu.sync_copy(sts_hbm.at[gbase], stsA)
        pltpu.sync_copy(h3_hbm.at[gbase + 1], hB)
        pltpu.sync_copy(av3_hbm.at[gbase + 1], avB)
        pltpu.sync_copy(sts_hbm.at[gbase + 1], stsB)
        plsc.subcore_barrier()
        for x0 in range(_D):
            pltpu.async_copy(tvt_hbm.at[avA.at[x0]], RB[x0], G[x0])

        def window(w, carry):
            for jj in range(_WIN):
                x = jj % _D
                sg = jj // 8
                srow = jj % 8
                h_c, av_c, sts_c = H[sg], AV[sg], STS[sg]
                rb, g = RB[x], G[x]

                if jj == 4:
                    @pl.when(w >= 1)
                    def _wb():
                        pltpu.make_async_copy(h3_hbm.at[gbase], hB, stg).wait()
                        pltpu.make_async_copy(av3_hbm.at[gbase], avB, stg).wait()
                        pltpu.make_async_copy(sts_hbm.at[gbase], stsB, stg).wait()
                if jj == 12:
                    @pl.when(w <= nwin - 2)
                    def _wa():
                        pltpu.make_async_copy(h3_hbm.at[gbase], hA, stg).wait()
                        pltpu.make_async_copy(av3_hbm.at[gbase], avA, stg).wait()
                        pltpu.make_async_copy(sts_hbm.at[gbase], stsA, stg).wait()

                pltpu.make_async_copy(tvt_hbm.at[av_c.at[srow]], rb, g).wait()

                def edge(e, ecarry):
                    ps = plsc.load_gather(
                        sts_c, [jnp.full((_L,), srow, jnp.int32),
                                jnp.zeros((_L,), jnp.int32) + e])
                    for cb in range(kd // _L):
                        sl = pl.ds(cb * _L, _L)
                        rb[e, sl] = (rb[e, sl] + rb[_BA + e, sl]) * ps
                    return ecarry
                lax.fori_loop(0, _BA, edge, 0)
                pltpu.sync_copy(rb.at[pl.ds(0, _BA)],
                                acc_sh.at[h_c.at[srow]], add=True)

                if jj <= _WIN - 1 - _D:
                    jj2 = jj + _D
                    av_n = AV[jj2 // 8]
                    srow2 = jj2 % 8
                    pltpu.async_copy(tvt_hbm.at[av_n.at[srow2]], rb, g)
                else:
                    srow2 = jj + _D - _WIN

                    @pl.when(w <= nwin - 2)
                    def _gnext():
                        pltpu.async_copy(tvt_hbm.at[avA.at[srow2]], rb, g)

                if jj == 0:
                    @pl.when(w >= 1)
                    def _sb():
                        gg = gbase + 2 * w + 1
                        pltpu.async_copy(h3_hbm.at[gg], hB, stg)
                        pltpu.async_copy(av3_hbm.at[gg], avB, stg)
                        pltpu.async_copy(sts_hbm.at[gg], stsB, stg)
                if jj == 8:
                    @pl.when(w <= nwin - 2)
                    def _sa():
                        gg = gbase + 2 * w + 2
                        pltpu.async_copy(h3_hbm.at[gg], hA, stg)
                        pltpu.async_copy(av3_hbm.at[gg], avA, stg)
                        pltpu.async_copy(sts_hbm.at[gg], stsA, stg)
            return carry

        lax.fori_loop(0, nwin, window, 0)
        plsc.subcore_barrier()

        @pl.when(sid == 0)
        def _flush():
            pltpu.sync_copy(acc_sh, out_hbm.at[c])

    return run(h3, av3, sts, tvt)


def _finish(acc, rs, ent):
    n, kd = ent.shape
    blk = 1000

    def body(acc_ref, rs_ref, ent_ref, o_ref):
        r = 1.0 / jnp.maximum(rs_ref[0] + rs_ref[1], 1e-30)
        x = (acc_ref[0] + acc_ref[1]) * r + ent_ref[...]
        o_ref[...] = jnp.where(x > 0, x, jnp.exp(x) - 1.0)

    return pl.pallas_call(
        body,
        grid=(n // blk,),
        in_specs=[
            pl.BlockSpec((2, blk, kd), lambda i: (0, i, 0)),
            pl.BlockSpec((2, blk, 1), lambda i: (0, i, 0)),
            pl.BlockSpec((blk, kd), lambda i: (i, 0)),
        ],
        out_specs=pl.BlockSpec((blk, kd), lambda i: (i, 0)),
        out_shape=jax.ShapeDtypeStruct((n, kd), jnp.float32),
    )(acc, rs, ent)


def kernel(attribute_triples, ent_feats, att_feats, val_feats, a_w, a_b, W):
    n, kd = ent_feats.shape
    e = attribute_triples.shape[0]
    real_rows = e // _B
    rows = _NW * 8 * ((real_rows + _NW * 8 - 1) // (_NW * 8))
    pad = rows * _B - e

    tri = attribute_triples.astype(jnp.int32)
    pad_h = (jnp.arange(pad, dtype=jnp.int32) * 16) % n
    pad_tri = jnp.stack([pad_h, jnp.zeros((pad,), jnp.int32),
                         jnp.zeros((pad,), jnp.int32)], axis=1)
    tri = jnp.concatenate([tri, pad_tri], axis=0)
    h2 = tri[:, 0].reshape(rows, _B)
    a2 = tri[:, 1].reshape(rows, _B)
    va2 = tri[:, 2].reshape(rows, _B)

    u, v, a_tab, v_tab = _dense_pre(
        ent_feats, att_feats, val_feats, a_w,
        a_b.reshape(1, 1).astype(jnp.float32), W)
    s2, rs = _sc_scores(h2, a2, u, v, n, real_rows)

    ngrp = rows * _B // (_GR * _BA)
    h3 = h2.reshape(ngrp * _GR, _BA).reshape(ngrp, _GR, _BA)
    av = jnp.concatenate(
        [a2.reshape(ngrp * _GR, _BA),
         va2.reshape(ngrp * _GR, _BA) + n], axis=1)
    av3 = av.reshape(ngrp, _GR, 2 * _BA)
    sts = s2.reshape(ngrp, _GR, _BA)
    tvt = jnp.concatenate([a_tab, v_tab], axis=0)

    acc = _sc_aggregate(h3, av3, sts, tvt, n, kd)
    return _finish(acc, rs.reshape(2, n, 1), ent_feats)


# speedup vs baseline: 1.0509x; 1.0509x over previous
import functools

import jax
import jax.numpy as jnp
from jax import lax
from jax.experimental import pallas as pl
from jax.experimental.pallas import tpu as pltpu
from jax.experimental.pallas import tpu_sc as plsc

_NC = 2
_NS = 16
_NW = _NC * _NS
_L = 16
_B = 128


def _dense_pre(ent, attf, valf, a_w, a_b2, W):
    n, kd = ent.shape
    vd = valf.shape[1]
    kh = kd // 2
    blk = 1000
    grid = n // blk

    def body(ent_ref, attf_ref, valf_ref, aw_ref, ab_ref, w_ref,
             u_ref, v_ref, a_out_ref, v_out_ref):
        aw = aw_ref[...]
        w = w_ref[...]
        zpad = jnp.zeros((kd, kd - 1), jnp.float32)
        rhs_u = jnp.concatenate([aw[:, :kd].T, zpad], axis=1)
        rhs_v = jnp.concatenate([aw[:, kd:].T, zpad], axis=1)
        resu = jnp.dot(ent_ref[...], rhs_u, preferred_element_type=jnp.float32)
        resv = jnp.dot(attf_ref[...], rhs_v, preferred_element_type=jnp.float32)
        u_ref[...] = resu[:, :1] + ab_ref[0, 0]
        v_ref[...] = resv[:, :1]
        a_out_ref[...] = jnp.dot(attf_ref[...], w[:kd, :],
                                 preferred_element_type=jnp.float32)
        v_out_ref[...] = jnp.dot(valf_ref[...], w[kd:, :],
                                 preferred_element_type=jnp.float32)

    out = pl.pallas_call(
        body,
        grid=(grid,),
        in_specs=[
            pl.BlockSpec((blk, kd), lambda i: (i, 0)),
            pl.BlockSpec((blk, kd), lambda i: (i, 0)),
            pl.BlockSpec((blk, vd), lambda i: (i, 0)),
            pl.BlockSpec((1, 2 * kd), lambda i: (0, 0)),
            pl.BlockSpec((1, 1), lambda i: (0, 0)),
            pl.BlockSpec((kd + vd, kd), lambda i: (0, 0)),
        ],
        out_specs=[
            pl.BlockSpec((blk, 1), lambda i: (i, 0)),
            pl.BlockSpec((blk, 1), lambda i: (i, 0)),
            pl.BlockSpec((blk, kd), lambda i: (i, 0)),
            pl.BlockSpec((blk, kd), lambda i: (i, 0)),
        ],
        out_shape=[
            jax.ShapeDtypeStruct((n, 1), jnp.float32),
            jax.ShapeDtypeStruct((n, 1), jnp.float32),
            jax.ShapeDtypeStruct((n, kd), jnp.float32),
            jax.ShapeDtypeStruct((n, kd), jnp.float32),
        ],
    )(ent, attf, valf, a_w, a_b2, W)
    u, v, a_tab, v_tab = out
    return u.reshape(n), v.reshape(n), a_tab, v_tab


def _sc_scores(h2, a2, u, v, n, real_rows):
    rows = h2.shape[0]
    rpt = rows // _NW
    mesh = plsc.VectorSubcoreMesh(core_axis_name="c", subcore_axis_name="s")

    @functools.partial(
        pl.kernel,
        out_type=(
            jax.ShapeDtypeStruct((rows, _B), jnp.float32),
            jax.ShapeDtypeStruct((_NC * n,), jnp.float32),
        ),
        mesh=mesh,
        scratch_types=[
            pltpu.VMEM((n,), jnp.float32),
            pltpu.VMEM((n,), jnp.float32),
            pltpu.VMEM((rpt, _B), jnp.int32),
            pltpu.VMEM((rpt, _B), jnp.int32),
            pltpu.VMEM((rpt, _B), jnp.float32),
            pltpu.VMEM_SHARED((n,), jnp.float32),
        ],
        compiler_params=pltpu.CompilerParams(needs_layout_passes=False),
    )
    def run(h2_hbm, a2_hbm, u_hbm, v_hbm, s2_hbm, rs_hbm,
            u_v, v_v, h_v, a_v, s_v, rs_sh):
        c = lax.axis_index("c")
        sid = lax.axis_index("s")
        wid = sid * _NC + c
        base = wid * rpt

        @pl.when(sid == 0)
        def _zero():
            def zloop(i, carry):
                u_v[pl.ds(i * _L, _L)] = jnp.zeros((_L,), jnp.float32)
                return carry
            lax.fori_loop(0, n // _L, zloop, 0)
            pltpu.sync_copy(u_v, rs_sh)

        pltpu.sync_copy(u_hbm, u_v)
        pltpu.sync_copy(v_hbm, v_v)
        pltpu.sync_copy(h2_hbm.at[pl.ds(base, rpt)], h_v)
        pltpu.sync_copy(a2_hbm.at[pl.ds(base, rpt)], a_v)
        plsc.subcore_barrier()

        def chunk(j, carry):
            @pl.when(base + j < real_rows)
            def _work():
                for k in range(_B // _L):
                    sl = pl.ds(k * _L, _L)
                    h16 = h_v[j, sl]
                    a16 = a_v[j, sl]
                    t = plsc.load_gather(u_v, [h16]) + plsc.load_gather(v_v, [a16])
                    t = jnp.where(t > 0, t, 0.2 * t)
                    s_v[j, sl] = jnp.exp(t)
                pltpu.sync_copy(s_v.at[j], rs_sh.at[h_v.at[j]], add=True)

            @pl.when(base + j >= real_rows)
            def _pad():
                for k in range(_B // _L):
                    s_v[j, pl.ds(k * _L, _L)] = jnp.zeros((_L,), jnp.float32)
            return carry

        lax.fori_loop(0, rpt, chunk, 0)
        pltpu.sync_copy(s_v, s2_hbm.at[pl.ds(base, rpt)])
        plsc.subcore_barrier()

        @pl.when(sid == 0)
        def _flush():
            pltpu.sync_copy(rs_sh, u_v)
            pltpu.sync_copy(u_v, rs_hbm.at[pl.ds(c * n, n)])

    return run(h2, a2, u, v)


_BA = 64
_GR = 8
_WIN = 16
_D = 2
_C1_SHARE = 0.30


def _sc_aggregate(h3, av3, sts, tvt, n, kd):
    ngrp = h3.shape[0]
    gpt2 = ngrp // _NS
    gpt1 = 2 * (int(gpt2 * _C1_SHARE) // 2)
    gpt0 = gpt2 - gpt1
    nwin0 = gpt0 // 2
    nwin1 = gpt1 // 2
    n_acc = _NS * 640 * ((n + _NS * 640 - 1) // (_NS * 640))
    zrows = n_acc // _NS
    mesh = plsc.VectorSubcoreMesh(core_axis_name="c", subcore_axis_name="s")

    @functools.partial(
        pl.kernel,
        out_type=jax.ShapeDtypeStruct((_NC, n_acc, kd), jnp.float32),
        mesh=mesh,
        scratch_types=[
            pltpu.VMEM((_GR, _BA), jnp.int32),
            pltpu.VMEM((_GR, _BA), jnp.int32),
            pltpu.VMEM((_GR, 2 * _BA), jnp.int32),
            pltpu.VMEM((_GR, 2 * _BA), jnp.int32),
            pltpu.VMEM((_GR, _BA), jnp.float32),
            pltpu.VMEM((_GR, _BA), jnp.float32),
            pltpu.VMEM((2 * _BA, kd), jnp.float32),
            pltpu.VMEM((2 * _BA, kd), jnp.float32),
            pltpu.VMEM_SHARED((n_acc, kd), jnp.float32),
            pltpu.SemaphoreType.DMA,
            pltpu.SemaphoreType.DMA,
            pltpu.SemaphoreType.DMA,
        ],
        compiler_params=pltpu.CompilerParams(needs_layout_passes=False),
    )
    def run(h3_hbm, av3_hbm, sts_hbm, tvt_hbm, out_hbm,
            hA, hB, avA, avB, stsA, stsB, rb0, rb1,
            acc_sh, g0, g1, stg):
        c = lax.axis_index("c")
        sid = lax.axis_index("s")
        gbase = jnp.where(c == 0, sid * gpt0, _NS * gpt0 + sid * gpt1)
        nwin = jnp.where(c == 0, nwin0, nwin1)

        H = (hA, hB)
        AV = (avA, avB)
        STS = (stsA, stsB)
        RB = (rb0, rb1)
        G = (g0, g1)

        def zfill(i, carry):
            for k in range(kd // _L):
                rb0[i, pl.ds(k * _L, _L)] = jnp.zeros((_L,), jnp.float32)
            return carry
        lax.fori_loop(0, 2 * _BA, zfill, 0)
        for q in range(zrows // (2 * _BA)):
            pltpu.sync_copy(
                rb0, acc_sh.at[pl.ds(sid * zrows + q * 2 * _BA, 2 * _BA)])

        pltpu.sync_copy(h3_hbm.at[gbase], hA)
        pltpu.sync_copy(av3_hbm.at[gbase], avA)
        pltpu.sync_copy(sts_hbm.at[gbase], stsA)
        pltpu.sync_copy(h3_hbm.at[gbase + 1], hB)
        pltpu.sync_copy(av3_hbm.at[gbase + 1], avB)
        pltpu.sync_copy(sts_hbm.at[gbase + 1], stsB)
        plsc.subcore_barrier()
        for x0 in range(_D):
            pltpu.async_copy(tvt_hbm.at[avA.at[x0]], RB[x0], G[x0])

        def window(w, carry):
            for jj in range(_WIN):
                x = jj % _D
                sg = jj // 8
                srow = jj % 8
                h_c, av_c, sts_c = H[sg], AV[sg], STS[sg]
                rb, g = RB[x], G[x]

                if jj == 4:
                    @pl.when(w >= 1)
                    def _wb():
                        pltpu.make_async_copy(h3_hbm.at[gbase], hB, stg).wait()
                        pltpu.make_async_copy(av3_hbm.at[gbase], avB, stg).wait()
                        pltpu.make_async_copy(sts_hbm.at[gbase], stsB, stg).wait()
                if jj == 12:
                    @pl.when(w <= nwin - 2)
                    def _wa():
                        pltpu.make_async_copy(h3_hbm.at[gbase], hA, stg).wait()
                        pltpu.make_async_copy(av3_hbm.at[gbase], avA, stg).wait()
                        pltpu.make_async_copy(sts_hbm.at[gbase], stsA, stg).wait()

                pltpu.make_async_copy(tvt_hbm.at[av_c.at[srow]], rb, g).wait()

                def edge(e, ecarry):
                    ps = plsc.load_gather(
                        sts_c, [jnp.full((_L,), srow, jnp.int32),
                                jnp.zeros((_L,), jnp.int32) + e])
                    for cb in range(kd // _L):
                        sl = pl.ds(cb * _L, _L)
                        rb[e, sl] = (rb[e, sl] + rb[_BA + e, sl]) * ps
                    return ecarry
                lax.fori_loop(0, _BA, edge, 0)
                pltpu.sync_copy(rb.at[pl.ds(0, _BA)],
                                acc_sh.at[h_c.at[srow]], add=True)

                if jj <= _WIN - 1 - _D:
                    jj2 = jj + _D
                    av_n = AV[jj2 // 8]
                    srow2 = jj2 % 8
                    pltpu.async_copy(tvt_hbm.at[av_n.at[srow2]], rb, g)
                else:
                    srow2 = jj + _D - _WIN

                    @pl.when(w <= nwin - 2)
                    def _gnext():
                        pltpu.async_copy(tvt_hbm.at[avA.at[srow2]], rb, g)

                if jj == 0:
                    @pl.when(w >= 1)
                    def _sb():
                        gg = gbase + 2 * w + 1
                        pltpu.async_copy(h3_hbm.at[gg], hB, stg)
                        pltpu.async_copy(av3_hbm.at[gg], avB, stg)
                        pltpu.async_copy(sts_hbm.at[gg], stsB, stg)
                if jj == 8:
                    @pl.when(w <= nwin - 2)
                    def _sa():
                        gg = gbase + 2 * w + 2
                        pltpu.async_copy(h3_hbm.at[gg], hA, stg)
                        pltpu.async_copy(av3_hbm.at[gg], avA, stg)
                        pltpu.async_copy(sts_hbm.at[gg], stsA, stg)
            return carry

        lax.fori_loop(0, nwin, window, 0)
        plsc.subcore_barrier()

        @pl.when(sid == 0)
        def _flush():
            pltpu.sync_copy(acc_sh, out_hbm.at[c])

    return run(h3, av3, sts, tvt)


def _finish(acc, rs, ent):
    n, kd = ent.shape
    blk = 1000

    def body(acc_ref, rs_ref, ent_ref, o_ref):
        r = 1.0 / jnp.maximum(rs_ref[0] + rs_ref[1], 1e-30)
        x = (acc_ref[0] + acc_ref[1]) * r + ent_ref[...]
        o_ref[...] = jnp.where(x > 0, x, jnp.exp(x) - 1.0)

    return pl.pallas_call(
        body,
        grid=(n // blk,),
        in_specs=[
            pl.BlockSpec((2, blk, kd), lambda i: (0, i, 0)),
            pl.BlockSpec((2, blk, 1), lambda i: (0, i, 0)),
            pl.BlockSpec((blk, kd), lambda i: (i, 0)),
        ],
        out_specs=pl.BlockSpec((blk, kd), lambda i: (i, 0)),
        out_shape=jax.ShapeDtypeStruct((n, kd), jnp.float32),
    )(acc, rs, ent)


def kernel(attribute_triples, ent_feats, att_feats, val_feats, a_w, a_b, W):
    n, kd = ent_feats.shape
    e = attribute_triples.shape[0]
    real_rows = e // _B
    rows = _NW * 8 * ((real_rows + _NW * 8 - 1) // (_NW * 8))
    pad = rows * _B - e

    tri = attribute_triples.astype(jnp.int32)
    pad_h = (jnp.arange(pad, dtype=jnp.int32) * 16) % n
    pad_tri = jnp.stack([pad_h, jnp.zeros((pad,), jnp.int32),
                         jnp.zeros((pad,), jnp.int32)], axis=1)
    tri = jnp.concatenate([tri, pad_tri], axis=0)
    h2 = tri[:, 0].reshape(rows, _B)
    a2 = tri[:, 1].reshape(rows, _B)
    va2 = tri[:, 2].reshape(rows, _B)

    u, v, a_tab, v_tab = _dense_pre(
        ent_feats, att_feats, val_feats, a_w,
        a_b.reshape(1, 1).astype(jnp.float32), W)
    s2, rs = _sc_scores(h2, a2, u, v, n, real_rows)

    ngrp = rows * _B // (_GR * _BA)
    h3 = h2.reshape(ngrp * _GR, _BA).reshape(ngrp, _GR, _BA)
    av = jnp.concatenate(
        [a2.reshape(ngrp * _GR, _BA),
         va2.reshape(ngrp * _GR, _BA) + n], axis=1)
    av3 = av.reshape(ngrp, _GR, 2 * _BA)
    sts = s2.reshape(ngrp, _GR, _BA)
    tvt = jnp.concatenate([a_tab, v_tab], axis=0)

    acc = _sc_aggregate(h3, av3, sts, tvt, n, kd)
    return _finish(acc, rs.reshape(2, n, 1), ent_feats)
